# baseline (speedup 1.0000x reference)
import functools

import jax
import jax.numpy as jnp
from jax import lax
from jax.experimental import pallas as pl
from jax.experimental.pallas import tpu as pltpu
from jax.experimental.pallas import tpu_sc as plsc

_CHUNK_TC = 1024
_CHUNK_SC = 128


def _enc_body(quant, freqs_ref, px_ref, py_ref, pz_ref, idx_ref, w_ref):
    f = pl.program_id(0)
    fq = freqs_ref[f]
    q = quant
    qf = float(q)

    ic = []
    wd = []
    for p_ref in (px_ref, py_ref, pz_ref):
        ang = p_ref[...] * fq
        ic_d, wd_d = [], []
        for a in range(2):
            t = jnp.sin(ang) if a == 0 else jnp.cos(ang)
            coord = (qf / 2.0) * t + (qf - 1.0) / 2.0
            i0f = jnp.floor(coord)
            frac = coord - i0f
            i0 = i0f.astype(jnp.int32)
            ic_a, wd_a = [], []
            for dx in range(2):
                xi = i0 + dx
                valid = ((xi >= 0) & (xi <= q - 1)).astype(jnp.float32)
                wgt = (frac if dx == 1 else 1.0 - frac) * valid
                ic_a.append(jnp.clip(xi, 0, q - 1))
                wd_a.append(wgt)
            ic_d.append(ic_a)
            wd_d.append(wd_a)
        ic.append(ic_d)
        wd.append(wd_d)

    vol = q * q * q
    for k in range(8):
        bx, by, bz = (k >> 2) & 1, (k >> 1) & 1, k & 1
        base = (f * 8 + k) * vol
        for dz in range(2):
            for dy in range(2):
                zy_i = ic[2][bz][dz] * (q * q) + ic[1][by][dy] * q
                zy_w = wd[2][bz][dz] * wd[1][by][dy]
                for dx in range(2):
                    j = dz * 4 + dy * 2 + dx
                    idx_ref[k, j, :] = base + zy_i + ic[0][bx][dx]
                    w_ref[k, j, :] = zy_w * wd[0][bx][dx] * 10.0


def _encode(px, py, pz, freqs, quant, interpret=False):
    n_pad = px.shape[0]
    nf = freqs.shape[0]
    nb = nf * 8
    grid = (nf, n_pad // _CHUNK_TC)
    return pl.pallas_call(
        functools.partial(_enc_body, quant),
        grid=grid,
        in_specs=[
            pl.BlockSpec(memory_space=pltpu.SMEM),
            pl.BlockSpec((_CHUNK_TC,), lambda f, c: (c,)),
            pl.BlockSpec((_CHUNK_TC,), lambda f, c: (c,)),
            pl.BlockSpec((_CHUNK_TC,), lambda f, c: (c,)),
        ],
        out_specs=[
            pl.BlockSpec((8, 8, _CHUNK_TC), lambda f, c: (f, 0, c)),
            pl.BlockSpec((8, 8, _CHUNK_TC), lambda f, c: (f, 0, c)),
        ],
        out_shape=[
            jax.ShapeDtypeStruct((nb, 8, n_pad), jnp.int32),
            jax.ShapeDtypeStruct((nb, 8, n_pad), jnp.float32),
        ],
        interpret=interpret,
    )(freqs, px, py, pz)


def _sc_body(nb, n_pad, idx_hbm, w_hbm, tab_hbm, out_hbm,
             idx_v, w_v, vals_v, feat_v, sem):
    wid = lax.axis_index("s") * 2 + lax.axis_index("c")
    b_per_w = nb // 32
    n_chunks = n_pad // _CHUNK_SC

    for bi in range(b_per_w):
        b = wid * b_per_w + bi

        def chunk_body(c, carry, b=b):
            off = pl.multiple_of(c * _CHUNK_SC, _CHUNK_SC)
            pltpu.sync_copy(idx_hbm.at[b, :, pl.ds(off, _CHUNK_SC)], idx_v)
            pltpu.sync_copy(w_hbm.at[b, :, pl.ds(off, _CHUNK_SC)], w_v)
            cps = [
                pltpu.async_copy(tab_hbm.at[idx_v.at[j]], vals_v.at[j], sem)
                for j in range(8)
            ]
            for cp in cps:
                cp.wait()
            for g in range(8):
                acc = jnp.zeros((16,), jnp.float32)
                for j in range(8):
                    v = vals_v[j, pl.ds(g * 16, 16)]
                    wv = w_v[j, pl.ds(g * 16, 16)]
                    acc = acc + v * wv
                feat_v[pl.ds(g * 16, 16)] = acc
            pltpu.sync_copy(feat_v, out_hbm.at[b, pl.ds(off, _CHUNK_SC)])
            return carry

        lax.fori_loop(0, n_chunks, chunk_body, 0)


def _sc_gather(idx, w, tab, nb, n_pad):
    kfn = functools.partial(
        pl.kernel,
        out_type=jax.ShapeDtypeStruct((nb, n_pad), jnp.float32),
        scratch_types=[
            pltpu.VMEM((8, _CHUNK_SC), jnp.int32),
            pltpu.VMEM((8, _CHUNK_SC), jnp.float32),
            pltpu.VMEM((8, _CHUNK_SC), jnp.float32),
            pltpu.VMEM((_CHUNK_SC,), jnp.float32),
            pltpu.SemaphoreType.DMA,
        ],
        mesh=plsc.VectorSubcoreMesh(core_axis_name="c", subcore_axis_name="s"),
    )(functools.partial(_sc_body, nb, n_pad))
    return kfn(idx, w, tab)


def kernel(points, cv, freqs):
    n = points.shape[0]
    nf = freqs.shape[0]
    nb = nf * 8
    quant = cv.shape[-1]
    n_pad = ((n + _CHUNK_TC - 1) // _CHUNK_TC) * _CHUNK_TC

    pts_pad = jnp.pad(points, ((0, n_pad - n), (0, 0)))
    px = pts_pad[:, 0]
    py = pts_pad[:, 1]
    pz = pts_pad[:, 2]

    idx, w = _encode(px, py, pz, freqs, quant)
    tab = cv.reshape(-1)
    feats = _sc_gather(idx, w, tab, nb, n_pad)
    return jnp.concatenate([points, feats[:, :n].T], axis=1)
